# baseline (speedup 1.0000x reference)
import functools

import jax
import jax.numpy as jnp
from jax import lax
from jax.experimental import pallas as pl
from jax.experimental.pallas import tpu as pltpu
from jax.experimental.pallas import tpu_sc as plsc

def _gather16(x, idx):
    dnums = lax.GatherDimensionNumbers(
        offset_dims=(), collapsed_slice_dims=(0,), start_index_map=(0,))
    return lax.gather(x, idx[:, None], dnums, slice_sizes=(1,),
                      mode=lax.GatherScatterMode.PROMISE_IN_BOUNDS)


N = 10000
E = 320000
HID = 128
H = 16
D = HID // H
L = 4

NC = 2
NS = 16
NW = NC * NS
EPW = E // NW
CH = 40
NCHUNK = EPW // CH
NP = 10240
RPT = NP // NS
ZR = 128
SPT = NP // 8 // NS

BN_ROWS = 1000
GRID = N // BN_ROWS

_HI = jax.lax.Precision.HIGHEST


def _edge_body(src_hbm, dst_hbm, el_hbm, er_hbm, feat_hbm, m_hbm,
               acc_out, s_out,
               isrc0, idst0, dsc0, xsc0, dmod0, el0, sb0, ft0,
               isrc1, idst1, dsc1, xsc1, dmod1, el1, sb1, ft1,
               m_v, acc_sh, s_sh,
               semi0, semg0, sems0, semi1, semg1, sems1):
    cid = lax.axis_index("c")
    sid = lax.axis_index("s")
    wid = cid * NS + sid

    zero16 = jnp.zeros((16,), jnp.float32)

    def zrow(i, carry):
        for j in range(8):
            ft0[i, pl.ds(16 * j, 16)] = zero16
        return carry

    lax.fori_loop(0, CH, zrow, 0)

    row0 = sid * RPT
    for k in range(RPT // CH):
        pltpu.sync_copy(ft0, acc_sh.at[pl.ds(row0 + k * CH, CH)])
    for k in range(SPT // CH):
        pltpu.sync_copy(ft0, s_sh.at[pl.ds(sid * SPT + k * CH, CH)])

    pltpu.sync_copy(m_hbm, m_v)
    plsc.subcore_barrier()

    mvec = m_v[:]
    half = lax.div(lax.iota(jnp.int32, 16), 8)
    ebase = wid * EPW

    P0 = (isrc0, idst0, dsc0, xsc0, dmod0, el0, sb0, ft0, semi0, semg0, sems0)
    P1 = (isrc1, idst1, dsc1, xsc1, dmod1, el1, sb1, ft1, semi1, semg1, sems1)

    def fire_idx(c, P):
        base = ebase + c * CH
        pltpu.async_copy(src_hbm.at[pl.ds(base, CH)], P[0], P[8])
        pltpu.async_copy(dst_hbm.at[pl.ds(base, CH)], P[1], P[8])

    def wait_idx(P):
        pltpu.make_async_copy(src_hbm.at[pl.ds(0, CH)], P[0], P[8]).wait()
        pltpu.make_async_copy(dst_hbm.at[pl.ds(0, CH)], P[1], P[8]).wait()

    def fire_gather(P):
        pltpu.async_copy(el_hbm.at[P[0]], P[5], P[9])
        pltpu.async_copy(er_hbm.at[P[1]], P[6], P[9])
        pltpu.async_copy(feat_hbm.at[P[0]], P[7], P[9])

    def wait_gather(P):
        pltpu.make_async_copy(el_hbm.at[P[0]], P[5], P[9]).wait()
        pltpu.make_async_copy(er_hbm.at[P[1]], P[6], P[9]).wait()
        pltpu.make_async_copy(feat_hbm.at[P[0]], P[7], P[9]).wait()

    def fire_scatter(P):
        pltpu.async_copy(P[6], s_sh.at[P[3]], P[10], add=True)
        pltpu.async_copy(P[7], acc_sh.at[P[2]], P[10], add=True)

    def wait_scatter(P):
        pltpu.make_async_copy(P[6], s_sh.at[P[3]], P[10]).wait()
        pltpu.make_async_copy(P[7], acc_sh.at[P[2]], P[10]).wait()

    def do_chunk(c, P, Q):
        wait_gather(P)
        for q0 in (0, 16, CH - 16):
            sl = pl.ds(q0, 16)
            dv = P[1][sl]
            P[2][sl] = dv
            P[3][sl] = lax.shift_right_logical(dv, 3)
            P[4][sl] = lax.rem(dv, 8)

        @pl.when(c + 2 < NCHUNK)
        def _():
            fire_idx(c + 2, P)

        @pl.when(jnp.logical_and(c + 1 < NCHUNK, c >= 1))
        def _():
            wait_scatter(Q)

        @pl.when(c + 1 < NCHUNK)
        def _():
            wait_idx(Q)
            fire_gather(Q)

        @plsc.parallel_loop(0, CH, 1, unroll=4)
        def edge(b):
            ev = P[5][b, pl.ds(0, 16)] + P[6][b, pl.ds(0, 16)]
            ev = jnp.where(ev >= 0, ev, 0.2 * ev) - mvec
            ex = jnp.exp(ev)
            dm = P[4][pl.ds(b, 16)][0]
            for k in range(8):
                P[6][b, pl.ds(16 * k, 16)] = jnp.where(dm == k, ex, zero16)
            for j in range(8):
                g = _gather16(ex, half + (2 * j))
                sl = pl.ds(16 * j, 16)
                P[7][b, sl] = P[7][b, sl] * g

        fire_scatter(P)

    fire_idx(0, P0)
    wait_idx(P0)
    fire_idx(1, P1)
    fire_gather(P0)

    def pair(t, carry):
        do_chunk(2 * t, P0, P1)
        do_chunk(2 * t + 1, P1, P0)
        return carry

    lax.fori_loop(0, NCHUNK // 2, pair, 0)

    wait_scatter(P0)
    wait_scatter(P1)
    plsc.subcore_barrier()

    for k in range(RPT // ZR):
        sl = pl.ds(row0 + k * ZR, ZR)
        pltpu.sync_copy(acc_sh.at[sl], acc_out.at[cid, sl])
    sl2 = pl.ds(sid * SPT, SPT)
    pltpu.sync_copy(s_sh.at[sl2], s_out.at[cid, sl2])


_edge_sc = functools.partial(
    pl.kernel,
    out_type=[
        jax.ShapeDtypeStruct((NC, NP, HID), jnp.float32),
        jax.ShapeDtypeStruct((NC, NP // 8, HID), jnp.float32),
    ],
    mesh=plsc.VectorSubcoreMesh(core_axis_name="c", subcore_axis_name="s"),
    compiler_params=pltpu.CompilerParams(use_tc_tiling_on_sc=False),
    scratch_types=(
        [pltpu.VMEM((CH,), jnp.int32)] * 4
        + [pltpu.VMEM((CH + 16,), jnp.int32)]
        + [pltpu.VMEM((CH, HID), jnp.float32)] * 3
    ) * 2 + [
        pltpu.VMEM((16,), jnp.float32),
        pltpu.VMEM_SHARED((NP, HID), jnp.float32),
        pltpu.VMEM_SHARED((NP // 8, HID), jnp.float32),
    ] + [pltpu.SemaphoreType.DMA] * 6,
)(_edge_body)


def _k_mm_stats(x_ref, w_ref, y_ref, stats_ref, acc):
    i = pl.program_id(0)
    y = jnp.dot(x_ref[...], w_ref[...], preferred_element_type=jnp.float32)
    y_ref[...] = y

    @pl.when(i == 0)
    def _():
        acc[...] = jnp.zeros_like(acc)

    s1 = jnp.sum(y, axis=0, keepdims=True)
    s2 = jnp.sum(y * y, axis=0, keepdims=True)
    pad = jnp.zeros((6, HID), jnp.float32)
    acc[...] += jnp.concatenate([s1, s2, pad], axis=0)

    @pl.when(i == pl.num_programs(0) - 1)
    def _():
        stats_ref[...] = acc[...]


def _mm_stats(x, w):
    return pl.pallas_call(
        _k_mm_stats,
        grid=(GRID,),
        in_specs=[
            pl.BlockSpec((BN_ROWS, x.shape[1]), lambda i: (i, 0)),
            pl.BlockSpec(w.shape, lambda i: (0, 0)),
        ],
        out_specs=[
            pl.BlockSpec((BN_ROWS, HID), lambda i: (i, 0)),
            pl.BlockSpec((8, HID), lambda i: (0, 0)),
        ],
        out_shape=[
            jax.ShapeDtypeStruct((N, HID), jnp.float32),
            jax.ShapeDtypeStruct((8, HID), jnp.float32),
        ],
        scratch_shapes=[pltpu.VMEM((8, HID), jnp.float32)],
    )(x, w)


def _bn_relu(y, stats, g, b):
    s = stats
    mu = s[0:1, :] * (1.0 / N)
    var = s[1:2, :] * (1.0 / N) - mu * mu
    return jnp.maximum(g * (y - mu) / jnp.sqrt(var + 1e-5) + b, 0.0)


def _el_er_m(feat, alf_ref, arf_ref, oneh_ref, el_ref, er_ref, m_ref,
             ml_acc, mr_acc):
    i = pl.program_id(0)
    oneh = oneh_ref[...]
    pad = jnp.zeros((feat.shape[0], HID - H), jnp.float32)
    el16 = jnp.dot(feat * alf_ref[...], oneh,
                   preferred_element_type=jnp.float32, precision=_HI)
    er16 = jnp.dot(feat * arf_ref[...], oneh,
                   preferred_element_type=jnp.float32, precision=_HI)
    el_ref[...] = jnp.concatenate([el16, pad], axis=1)
    er_ref[...] = jnp.concatenate([er16, pad], axis=1)

    @pl.when(i == 0)
    def _():
        ml_acc[...] = jnp.full((8, H), -jnp.inf, jnp.float32)
        mr_acc[...] = jnp.full((8, H), -jnp.inf, jnp.float32)

    bl = jnp.broadcast_to(jnp.max(el16, axis=0, keepdims=True), (8, H))
    br = jnp.broadcast_to(jnp.max(er16, axis=0, keepdims=True), (8, H))
    ml_acc[...] = jnp.maximum(ml_acc[...], bl)
    mr_acc[...] = jnp.maximum(mr_acc[...], br)

    @pl.when(i == pl.num_programs(0) - 1)
    def _():
        m_ref[...] = jnp.maximum(ml_acc[...] + mr_acc[...], 0.0)


def _k_embed_apply(y_ref, stats_ref, ge_ref, be_ref, w1_ref, wg_ref,
                   alf_ref, arf_ref, oneh_ref,
                   h_ref, feat_ref, el_ref, er_ref, m_ref, ml_acc, mr_acc):
    t = _bn_relu(y_ref[...], stats_ref[...], ge_ref[...], be_ref[...])
    h = jnp.dot(t, w1_ref[...], preferred_element_type=jnp.float32) + t
    feat = jnp.dot(h, wg_ref[...], preferred_element_type=jnp.float32)
    h_ref[...] = h
    feat_ref[...] = feat
    _el_er_m(feat, alf_ref, arf_ref, oneh_ref, el_ref, er_ref, m_ref,
             ml_acc, mr_acc)


def _embed_apply(y, stats, ge, be, w1, wg, alf, arf, oneh):
    full = lambda a: pl.BlockSpec(a.shape, lambda i: tuple(0 for _ in a.shape))
    return pl.pallas_call(
        _k_embed_apply,
        grid=(GRID,),
        in_specs=[
            pl.BlockSpec((BN_ROWS, HID), lambda i: (i, 0)),
            full(stats), full(ge), full(be), full(w1), full(wg),
            full(alf), full(arf), full(oneh),
        ],
        out_specs=[
            pl.BlockSpec((BN_ROWS, HID), lambda i: (i, 0)),
            pl.BlockSpec((BN_ROWS, HID), lambda i: (i, 0)),
            pl.BlockSpec((BN_ROWS, HID), lambda i: (i, 0)),
            pl.BlockSpec((BN_ROWS, HID), lambda i: (i, 0)),
            pl.BlockSpec((8, H), lambda i: (0, 0)),
        ],
        out_shape=[
            jax.ShapeDtypeStruct((N, HID), jnp.float32),
            jax.ShapeDtypeStruct((N, HID), jnp.float32),
            jax.ShapeDtypeStruct((N, HID), jnp.float32),
            jax.ShapeDtypeStruct((N, HID), jnp.float32),
            jax.ShapeDtypeStruct((8, H), jnp.float32),
        ],
        scratch_shapes=[pltpu.VMEM((8, H), jnp.float32),
                        pltpu.VMEM((8, H), jnp.float32)],
    )(y, stats, ge, be, w1, wg, alf, arf, oneh)


def _gat_update(acc_ref, ssum_ref, h_ref, bg_ref, eexp_ref):
    a = acc_ref[0] + acc_ref[1]
    s = ssum_ref[0] + ssum_ref[1]
    sexp = jnp.dot(s, eexp_ref[...], preferred_element_type=jnp.float32,
                   precision=_HI)
    r = jnp.where(sexp > 0.0, a / jnp.where(sexp > 0.0, sexp, 1.0), 0.0)
    r = r + bg_ref[...]
    r = jnp.where(r >= 0, r, 0.01 * r)
    return r + h_ref[...]


def _k_mid(acc_ref, ssum_ref, h_ref, bg_ref, eexp_ref, wg_ref,
           alf_ref, arf_ref, oneh_ref,
           h2_ref, feat_ref, el_ref, er_ref, m_ref, ml_acc, mr_acc):
    h = _gat_update(acc_ref, ssum_ref, h_ref, bg_ref, eexp_ref)
    feat = jnp.dot(h, wg_ref[...], preferred_element_type=jnp.float32)
    h2_ref[...] = h
    feat_ref[...] = feat
    _el_er_m(feat, alf_ref, arf_ref, oneh_ref, el_ref, er_ref, m_ref,
             ml_acc, mr_acc)


def _mid(acc, ssum, h, bg_i, eexp, wg, alf, arf, oneh):
    full = lambda a: pl.BlockSpec(a.shape, lambda i: tuple(0 for _ in a.shape))
    return pl.pallas_call(
        _k_mid,
        grid=(GRID,),
        in_specs=[
            pl.BlockSpec((NC, BN_ROWS, HID), lambda i: (0, i, 0)),
            pl.BlockSpec((NC, BN_ROWS, H), lambda i: (0, i, 0)),
            pl.BlockSpec((BN_ROWS, HID), lambda i: (i, 0)),
            full(bg_i), full(eexp), full(wg), full(alf), full(arf), full(oneh),
        ],
        out_specs=[
            pl.BlockSpec((BN_ROWS, HID), lambda i: (i, 0)),
            pl.BlockSpec((BN_ROWS, HID), lambda i: (i, 0)),
            pl.BlockSpec((BN_ROWS, HID), lambda i: (i, 0)),
            pl.BlockSpec((BN_ROWS, HID), lambda i: (i, 0)),
            pl.BlockSpec((8, H), lambda i: (0, 0)),
        ],
        out_shape=[
            jax.ShapeDtypeStruct((N, HID), jnp.float32),
            jax.ShapeDtypeStruct((N, HID), jnp.float32),
            jax.ShapeDtypeStruct((N, HID), jnp.float32),
            jax.ShapeDtypeStruct((N, HID), jnp.float32),
            jax.ShapeDtypeStruct((8, H), jnp.float32),
        ],
        scratch_shapes=[pltpu.VMEM((8, H), jnp.float32),
                        pltpu.VMEM((8, H), jnp.float32)],
    )(acc, ssum, h, bg_i, eexp, wg, alf, arf, oneh)


def _k_last(acc_ref, ssum_ref, h_ref, bg_ref, eexp_ref, w0d_ref,
            y2_ref, stats_ref, accsc):
    i = pl.program_id(0)
    h = _gat_update(acc_ref, ssum_ref, h_ref, bg_ref, eexp_ref)
    y2 = jnp.dot(h, w0d_ref[...], preferred_element_type=jnp.float32)
    y2_ref[...] = y2

    @pl.when(i == 0)
    def _():
        accsc[...] = jnp.zeros_like(accsc)

    s1 = jnp.sum(y2, axis=0, keepdims=True)
    s2 = jnp.sum(y2 * y2, axis=0, keepdims=True)
    pad = jnp.zeros((6, HID), jnp.float32)
    accsc[...] += jnp.concatenate([s1, s2, pad], axis=0)

    @pl.when(i == pl.num_programs(0) - 1)
    def _():
        stats_ref[...] = accsc[...]


def _last(acc, ssum, h, bg_i, eexp, w0d):
    full = lambda a: pl.BlockSpec(a.shape, lambda i: tuple(0 for _ in a.shape))
    return pl.pallas_call(
        _k_last,
        grid=(GRID,),
        in_specs=[
            pl.BlockSpec((NC, BN_ROWS, HID), lambda i: (0, i, 0)),
            pl.BlockSpec((NC, BN_ROWS, H), lambda i: (0, i, 0)),
            pl.BlockSpec((BN_ROWS, HID), lambda i: (i, 0)),
            full(bg_i), full(eexp), full(w0d),
        ],
        out_specs=[
            pl.BlockSpec((BN_ROWS, HID), lambda i: (i, 0)),
            pl.BlockSpec((8, HID), lambda i: (0, 0)),
        ],
        out_shape=[
            jax.ShapeDtypeStruct((N, HID), jnp.float32),
            jax.ShapeDtypeStruct((8, HID), jnp.float32),
        ],
        scratch_shapes=[pltpu.VMEM((8, HID), jnp.float32)],
    )(acc, ssum, h, bg_i, eexp, w0d)


def _k_fin(y2_ref, stats_ref, gd_ref, bd_ref, w1d_ref, o_ref):
    t2 = _bn_relu(y2_ref[...], stats_ref[...], gd_ref[...], bd_ref[...])
    o_ref[...] = jnp.dot(t2, w1d_ref[...], preferred_element_type=jnp.float32)


def _fin(y2, stats, gd, bd, w1d):
    full = lambda a: pl.BlockSpec(a.shape, lambda i: tuple(0 for _ in a.shape))
    return pl.pallas_call(
        _k_fin,
        grid=(GRID,),
        in_specs=[
            pl.BlockSpec((BN_ROWS, HID), lambda i: (i, 0)),
            full(stats), full(gd), full(bd), full(w1d),
        ],
        out_specs=pl.BlockSpec((BN_ROWS, HID), lambda i: (i, 0)),
        out_shape=jax.ShapeDtypeStruct((N, HID), jnp.float32),
    )(y2, stats, gd, bd, w1d)


def kernel(x, edge_index, W0e, W1e, ge, be, Wg, al, ar, bg, W0d, W1d, gd, bd):
    src = edge_index[0]
    dst = edge_index[1]

    k128 = jnp.arange(HID) // D
    oneh = (k128[:, None] == jnp.arange(H)[None, :]).astype(jnp.float32)
    eexp = (jnp.arange(H)[:, None] == k128[None, :]).astype(jnp.float32)

    ge2 = ge.reshape(1, HID)
    be2 = be.reshape(1, HID)
    gd2 = gd.reshape(1, HID)
    bd2 = bd.reshape(1, HID)

    y, stats = _mm_stats(x, W0e)
    h, feat, el, er, m = _embed_apply(
        y, stats, ge2, be2, W1e, Wg[0],
        al[0].reshape(1, HID), ar[0].reshape(1, HID), oneh)

    for i in range(L):
        acc, spack = _edge_sc(src, dst, el, er, feat, m[0])
        ssum = spack.reshape(NC, NP, H)
        if i < L - 1:
            h, feat, el, er, m = _mid(
                acc, ssum, h, bg[i].reshape(1, HID), eexp, Wg[i + 1],
                al[i + 1].reshape(1, HID), ar[i + 1].reshape(1, HID), oneh)
        else:
            y2, stats2 = _last(acc, ssum, h, bg[i].reshape(1, HID), eexp, W0d)

    return _fin(y2, stats2, gd2, bd2, W1d)


# speedup vs baseline: 1.0030x; 1.0030x over previous
import functools

import jax
import jax.numpy as jnp
from jax import lax
from jax.experimental import pallas as pl
from jax.experimental.pallas import tpu as pltpu
from jax.experimental.pallas import tpu_sc as plsc

def _gather16(x, idx):
    dnums = lax.GatherDimensionNumbers(
        offset_dims=(), collapsed_slice_dims=(0,), start_index_map=(0,))
    return lax.gather(x, idx[:, None], dnums, slice_sizes=(1,),
                      mode=lax.GatherScatterMode.PROMISE_IN_BOUNDS)


N = 10000
E = 320000
HID = 128
H = 16
D = HID // H
L = 4

NC = 2
NS = 16
NW = NC * NS
EPW = E // NW
CH = 40
NCHUNK = EPW // CH
NP = 10240
RPT = NP // NS
ZR = 128
SPT = NP // 8 // NS

BN_ROWS = 1000
GRID = N // BN_ROWS

_HI = jax.lax.Precision.HIGHEST


def _edge_body(src_hbm, dst_hbm, el_hbm, er_hbm, feat_hbm, m_hbm,
               acc_out, s_out,
               isrc0, idst0, dsc0, xsc0, dmod0, el0, sb0, ft0,
               isrc1, idst1, dsc1, xsc1, dmod1, el1, sb1, ft1,
               m_v, acc_sh, s_sh,
               semi0, semg0, sems0, semi1, semg1, sems1):
    cid = lax.axis_index("c")
    sid = lax.axis_index("s")
    wid = cid * NS + sid

    zero16 = jnp.zeros((16,), jnp.float32)

    def zrow(i, carry):
        for j in range(8):
            ft0[i, pl.ds(16 * j, 16)] = zero16
        return carry

    lax.fori_loop(0, CH, zrow, 0)

    row0 = sid * RPT
    for k in range(RPT // CH):
        pltpu.sync_copy(ft0, acc_sh.at[pl.ds(row0 + k * CH, CH)])
    for k in range(SPT // CH):
        pltpu.sync_copy(ft0, s_sh.at[pl.ds(sid * SPT + k * CH, CH)])

    pltpu.sync_copy(m_hbm, m_v)
    plsc.subcore_barrier()

    mvec = m_v[:]
    half = lax.div(lax.iota(jnp.int32, 16), 8)
    ebase = wid * EPW

    P0 = (isrc0, idst0, dsc0, xsc0, dmod0, el0, sb0, ft0, semi0, semg0, sems0)
    P1 = (isrc1, idst1, dsc1, xsc1, dmod1, el1, sb1, ft1, semi1, semg1, sems1)

    def fire_idx(c, P):
        base = ebase + c * CH
        pltpu.async_copy(src_hbm.at[pl.ds(base, CH)], P[0], P[8])
        pltpu.async_copy(dst_hbm.at[pl.ds(base, CH)], P[1], P[8])

    def wait_idx(P):
        pltpu.make_async_copy(src_hbm.at[pl.ds(0, CH)], P[0], P[8]).wait()
        pltpu.make_async_copy(dst_hbm.at[pl.ds(0, CH)], P[1], P[8]).wait()

    def fire_gather(P):
        pltpu.async_copy(el_hbm.at[P[0]], P[5], P[9])
        pltpu.async_copy(er_hbm.at[P[1]], P[6], P[9])
        pltpu.async_copy(feat_hbm.at[P[0]], P[7], P[9])

    def wait_gather(P):
        pltpu.make_async_copy(el_hbm.at[P[0]], P[5], P[9]).wait()
        pltpu.make_async_copy(er_hbm.at[P[1]], P[6], P[9]).wait()
        pltpu.make_async_copy(feat_hbm.at[P[0]], P[7], P[9]).wait()

    def fire_scatter(P):
        pltpu.async_copy(P[6], s_sh.at[P[3]], P[10], add=True)
        pltpu.async_copy(P[7], acc_sh.at[P[2]], P[10], add=True)

    def wait_scatter(P):
        pltpu.make_async_copy(P[6], s_sh.at[P[3]], P[10]).wait()
        pltpu.make_async_copy(P[7], acc_sh.at[P[2]], P[10]).wait()

    def do_chunk(c, P, Q):
        wait_gather(P)
        for q0 in (0, 16, CH - 16):
            sl = pl.ds(q0, 16)
            dv = P[1][sl]
            P[2][sl] = dv
            P[3][sl] = lax.shift_right_logical(dv, 3)
            P[4][sl] = lax.rem(dv, 8)

        @pl.when(c + 2 < NCHUNK)
        def _():
            fire_idx(c + 2, P)

        @pl.when(jnp.logical_and(c + 1 < NCHUNK, c >= 1))
        def _():
            wait_scatter(Q)

        @pl.when(c + 1 < NCHUNK)
        def _():
            wait_idx(Q)
            fire_gather(Q)

        @plsc.parallel_loop(0, CH, 1, unroll=2)
        def edge(b):
            ev = P[5][b, pl.ds(0, 16)] + P[6][b, pl.ds(0, 16)]
            ev = jnp.where(ev >= 0, ev, 0.2 * ev) - mvec
            ex = jnp.exp(ev)
            dm = P[4][pl.ds(b, 16)][0]
            for k in range(8):
                P[6][b, pl.ds(16 * k, 16)] = jnp.where(dm == k, ex, zero16)
            for j in range(8):
                g = _gather16(ex, half + (2 * j))
                sl = pl.ds(16 * j, 16)
                P[7][b, sl] = P[7][b, sl] * g

        fire_scatter(P)

    fire_idx(0, P0)
    wait_idx(P0)
    fire_idx(1, P1)
    fire_gather(P0)

    def pair(t, carry):
        do_chunk(2 * t, P0, P1)
        do_chunk(2 * t + 1, P1, P0)
        return carry

    lax.fori_loop(0, NCHUNK // 2, pair, 0)

    wait_scatter(P0)
    wait_scatter(P1)
    plsc.subcore_barrier()

    for k in range(RPT // ZR):
        sl = pl.ds(row0 + k * ZR, ZR)
        pltpu.sync_copy(acc_sh.at[sl], acc_out.at[cid, sl])
    sl2 = pl.ds(sid * SPT, SPT)
    pltpu.sync_copy(s_sh.at[sl2], s_out.at[cid, sl2])


_edge_sc = functools.partial(
    pl.kernel,
    out_type=[
        jax.ShapeDtypeStruct((NC, NP, HID), jnp.float32),
        jax.ShapeDtypeStruct((NC, NP // 8, HID), jnp.float32),
    ],
    mesh=plsc.VectorSubcoreMesh(core_axis_name="c", subcore_axis_name="s"),
    compiler_params=pltpu.CompilerParams(use_tc_tiling_on_sc=False),
    scratch_types=(
        [pltpu.VMEM((CH,), jnp.int32)] * 4
        + [pltpu.VMEM((CH + 16,), jnp.int32)]
        + [pltpu.VMEM((CH, HID), jnp.float32)] * 3
    ) * 2 + [
        pltpu.VMEM((16,), jnp.float32),
        pltpu.VMEM_SHARED((NP, HID), jnp.float32),
        pltpu.VMEM_SHARED((NP // 8, HID), jnp.float32),
    ] + [pltpu.SemaphoreType.DMA] * 6,
)(_edge_body)


def _k_mm_stats(x_ref, w_ref, y_ref, stats_ref, acc):
    i = pl.program_id(0)
    y = jnp.dot(x_ref[...], w_ref[...], preferred_element_type=jnp.float32)
    y_ref[...] = y

    @pl.when(i == 0)
    def _():
        acc[...] = jnp.zeros_like(acc)

    s1 = jnp.sum(y, axis=0, keepdims=True)
    s2 = jnp.sum(y * y, axis=0, keepdims=True)
    pad = jnp.zeros((6, HID), jnp.float32)
    acc[...] += jnp.concatenate([s1, s2, pad], axis=0)

    @pl.when(i == pl.num_programs(0) - 1)
    def _():
        stats_ref[...] = acc[...]


def _mm_stats(x, w):
    return pl.pallas_call(
        _k_mm_stats,
        grid=(GRID,),
        in_specs=[
            pl.BlockSpec((BN_ROWS, x.shape[1]), lambda i: (i, 0)),
            pl.BlockSpec(w.shape, lambda i: (0, 0)),
        ],
        out_specs=[
            pl.BlockSpec((BN_ROWS, HID), lambda i: (i, 0)),
            pl.BlockSpec((8, HID), lambda i: (0, 0)),
        ],
        out_shape=[
            jax.ShapeDtypeStruct((N, HID), jnp.float32),
            jax.ShapeDtypeStruct((8, HID), jnp.float32),
        ],
        scratch_shapes=[pltpu.VMEM((8, HID), jnp.float32)],
    )(x, w)


def _bn_relu(y, stats, g, b):
    s = stats
    mu = s[0:1, :] * (1.0 / N)
    var = s[1:2, :] * (1.0 / N) - mu * mu
    return jnp.maximum(g * (y - mu) / jnp.sqrt(var + 1e-5) + b, 0.0)


def _el_er_m(feat, alf_ref, arf_ref, oneh_ref, el_ref, er_ref, m_ref,
             ml_acc, mr_acc):
    i = pl.program_id(0)
    oneh = oneh_ref[...]
    pad = jnp.zeros((feat.shape[0], HID - H), jnp.float32)
    el16 = jnp.dot(feat * alf_ref[...], oneh,
                   preferred_element_type=jnp.float32, precision=_HI)
    er16 = jnp.dot(feat * arf_ref[...], oneh,
                   preferred_element_type=jnp.float32, precision=_HI)
    el_ref[...] = jnp.concatenate([el16, pad], axis=1)
    er_ref[...] = jnp.concatenate([er16, pad], axis=1)

    @pl.when(i == 0)
    def _():
        ml_acc[...] = jnp.full((8, H), -jnp.inf, jnp.float32)
        mr_acc[...] = jnp.full((8, H), -jnp.inf, jnp.float32)

    bl = jnp.broadcast_to(jnp.max(el16, axis=0, keepdims=True), (8, H))
    br = jnp.broadcast_to(jnp.max(er16, axis=0, keepdims=True), (8, H))
    ml_acc[...] = jnp.maximum(ml_acc[...], bl)
    mr_acc[...] = jnp.maximum(mr_acc[...], br)

    @pl.when(i == pl.num_programs(0) - 1)
    def _():
        m_ref[...] = jnp.maximum(ml_acc[...] + mr_acc[...], 0.0)


def _k_embed_apply(y_ref, stats_ref, ge_ref, be_ref, w1_ref, wg_ref,
                   alf_ref, arf_ref, oneh_ref,
                   h_ref, feat_ref, el_ref, er_ref, m_ref, ml_acc, mr_acc):
    t = _bn_relu(y_ref[...], stats_ref[...], ge_ref[...], be_ref[...])
    h = jnp.dot(t, w1_ref[...], preferred_element_type=jnp.float32) + t
    feat = jnp.dot(h, wg_ref[...], preferred_element_type=jnp.float32)
    h_ref[...] = h
    feat_ref[...] = feat
    _el_er_m(feat, alf_ref, arf_ref, oneh_ref, el_ref, er_ref, m_ref,
             ml_acc, mr_acc)


def _embed_apply(y, stats, ge, be, w1, wg, alf, arf, oneh):
    full = lambda a: pl.BlockSpec(a.shape, lambda i: tuple(0 for _ in a.shape))
    return pl.pallas_call(
        _k_embed_apply,
        grid=(GRID,),
        in_specs=[
            pl.BlockSpec((BN_ROWS, HID), lambda i: (i, 0)),
            full(stats), full(ge), full(be), full(w1), full(wg),
            full(alf), full(arf), full(oneh),
        ],
        out_specs=[
            pl.BlockSpec((BN_ROWS, HID), lambda i: (i, 0)),
            pl.BlockSpec((BN_ROWS, HID), lambda i: (i, 0)),
            pl.BlockSpec((BN_ROWS, HID), lambda i: (i, 0)),
            pl.BlockSpec((BN_ROWS, HID), lambda i: (i, 0)),
            pl.BlockSpec((8, H), lambda i: (0, 0)),
        ],
        out_shape=[
            jax.ShapeDtypeStruct((N, HID), jnp.float32),
            jax.ShapeDtypeStruct((N, HID), jnp.float32),
            jax.ShapeDtypeStruct((N, HID), jnp.float32),
            jax.ShapeDtypeStruct((N, HID), jnp.float32),
            jax.ShapeDtypeStruct((8, H), jnp.float32),
        ],
        scratch_shapes=[pltpu.VMEM((8, H), jnp.float32),
                        pltpu.VMEM((8, H), jnp.float32)],
    )(y, stats, ge, be, w1, wg, alf, arf, oneh)


def _gat_update(acc_ref, ssum_ref, h_ref, bg_ref, eexp_ref):
    a = acc_ref[0] + acc_ref[1]
    s = ssum_ref[0] + ssum_ref[1]
    sexp = jnp.dot(s, eexp_ref[...], preferred_element_type=jnp.float32,
                   precision=_HI)
    r = jnp.where(sexp > 0.0, a / jnp.where(sexp > 0.0, sexp, 1.0), 0.0)
    r = r + bg_ref[...]
    r = jnp.where(r >= 0, r, 0.01 * r)
    return r + h_ref[...]


def _k_mid(acc_ref, ssum_ref, h_ref, bg_ref, eexp_ref, wg_ref,
           alf_ref, arf_ref, oneh_ref,
           h2_ref, feat_ref, el_ref, er_ref, m_ref, ml_acc, mr_acc):
    h = _gat_update(acc_ref, ssum_ref, h_ref, bg_ref, eexp_ref)
    feat = jnp.dot(h, wg_ref[...], preferred_element_type=jnp.float32)
    h2_ref[...] = h
    feat_ref[...] = feat
    _el_er_m(feat, alf_ref, arf_ref, oneh_ref, el_ref, er_ref, m_ref,
             ml_acc, mr_acc)


def _mid(acc, ssum, h, bg_i, eexp, wg, alf, arf, oneh):
    full = lambda a: pl.BlockSpec(a.shape, lambda i: tuple(0 for _ in a.shape))
    return pl.pallas_call(
        _k_mid,
        grid=(GRID,),
        in_specs=[
            pl.BlockSpec((NC, BN_ROWS, HID), lambda i: (0, i, 0)),
            pl.BlockSpec((NC, BN_ROWS, H), lambda i: (0, i, 0)),
            pl.BlockSpec((BN_ROWS, HID), lambda i: (i, 0)),
            full(bg_i), full(eexp), full(wg), full(alf), full(arf), full(oneh),
        ],
        out_specs=[
            pl.BlockSpec((BN_ROWS, HID), lambda i: (i, 0)),
            pl.BlockSpec((BN_ROWS, HID), lambda i: (i, 0)),
            pl.BlockSpec((BN_ROWS, HID), lambda i: (i, 0)),
            pl.BlockSpec((BN_ROWS, HID), lambda i: (i, 0)),
            pl.BlockSpec((8, H), lambda i: (0, 0)),
        ],
        out_shape=[
            jax.ShapeDtypeStruct((N, HID), jnp.float32),
            jax.ShapeDtypeStruct((N, HID), jnp.float32),
            jax.ShapeDtypeStruct((N, HID), jnp.float32),
            jax.ShapeDtypeStruct((N, HID), jnp.float32),
            jax.ShapeDtypeStruct((8, H), jnp.float32),
        ],
        scratch_shapes=[pltpu.VMEM((8, H), jnp.float32),
                        pltpu.VMEM((8, H), jnp.float32)],
    )(acc, ssum, h, bg_i, eexp, wg, alf, arf, oneh)


def _k_last(acc_ref, ssum_ref, h_ref, bg_ref, eexp_ref, w0d_ref,
            y2_ref, stats_ref, accsc):
    i = pl.program_id(0)
    h = _gat_update(acc_ref, ssum_ref, h_ref, bg_ref, eexp_ref)
    y2 = jnp.dot(h, w0d_ref[...], preferred_element_type=jnp.float32)
    y2_ref[...] = y2

    @pl.when(i == 0)
    def _():
        accsc[...] = jnp.zeros_like(accsc)

    s1 = jnp.sum(y2, axis=0, keepdims=True)
    s2 = jnp.sum(y2 * y2, axis=0, keepdims=True)
    pad = jnp.zeros((6, HID), jnp.float32)
    accsc[...] += jnp.concatenate([s1, s2, pad], axis=0)

    @pl.when(i == pl.num_programs(0) - 1)
    def _():
        stats_ref[...] = accsc[...]


def _last(acc, ssum, h, bg_i, eexp, w0d):
    full = lambda a: pl.BlockSpec(a.shape, lambda i: tuple(0 for _ in a.shape))
    return pl.pallas_call(
        _k_last,
        grid=(GRID,),
        in_specs=[
            pl.BlockSpec((NC, BN_ROWS, HID), lambda i: (0, i, 0)),
            pl.BlockSpec((NC, BN_ROWS, H), lambda i: (0, i, 0)),
            pl.BlockSpec((BN_ROWS, HID), lambda i: (i, 0)),
            full(bg_i), full(eexp), full(w0d),
        ],
        out_specs=[
            pl.BlockSpec((BN_ROWS, HID), lambda i: (i, 0)),
            pl.BlockSpec((8, HID), lambda i: (0, 0)),
        ],
        out_shape=[
            jax.ShapeDtypeStruct((N, HID), jnp.float32),
            jax.ShapeDtypeStruct((8, HID), jnp.float32),
        ],
        scratch_shapes=[pltpu.VMEM((8, HID), jnp.float32)],
    )(acc, ssum, h, bg_i, eexp, w0d)


def _k_fin(y2_ref, stats_ref, gd_ref, bd_ref, w1d_ref, o_ref):
    t2 = _bn_relu(y2_ref[...], stats_ref[...], gd_ref[...], bd_ref[...])
    o_ref[...] = jnp.dot(t2, w1d_ref[...], preferred_element_type=jnp.float32)


def _fin(y2, stats, gd, bd, w1d):
    full = lambda a: pl.BlockSpec(a.shape, lambda i: tuple(0 for _ in a.shape))
    return pl.pallas_call(
        _k_fin,
        grid=(GRID,),
        in_specs=[
            pl.BlockSpec((BN_ROWS, HID), lambda i: (i, 0)),
            full(stats), full(gd), full(bd), full(w1d),
        ],
        out_specs=pl.BlockSpec((BN_ROWS, HID), lambda i: (i, 0)),
        out_shape=jax.ShapeDtypeStruct((N, HID), jnp.float32),
    )(y2, stats, gd, bd, w1d)


def kernel(x, edge_index, W0e, W1e, ge, be, Wg, al, ar, bg, W0d, W1d, gd, bd):
    src = edge_index[0]
    dst = edge_index[1]

    k128 = jnp.arange(HID) // D
    oneh = (k128[:, None] == jnp.arange(H)[None, :]).astype(jnp.float32)
    eexp = (jnp.arange(H)[:, None] == k128[None, :]).astype(jnp.float32)

    ge2 = ge.reshape(1, HID)
    be2 = be.reshape(1, HID)
    gd2 = gd.reshape(1, HID)
    bd2 = bd.reshape(1, HID)

    y, stats = _mm_stats(x, W0e)
    h, feat, el, er, m = _embed_apply(
        y, stats, ge2, be2, W1e, Wg[0],
        al[0].reshape(1, HID), ar[0].reshape(1, HID), oneh)

    for i in range(L):
        acc, spack = _edge_sc(src, dst, el, er, feat, m[0])
        ssum = spack.reshape(NC, NP, H)
        if i < L - 1:
            h, feat, el, er, m = _mid(
                acc, ssum, h, bg[i].reshape(1, HID), eexp, Wg[i + 1],
                al[i + 1].reshape(1, HID), ar[i + 1].reshape(1, HID), oneh)
        else:
            y2, stats2 = _last(acc, ssum, h, bg[i].reshape(1, HID), eexp, W0d)

    return _fin(y2, stats2, gd2, bd2, W1d)
